# baseline (speedup 1.0000x reference)
import functools

import jax
import jax.numpy as jnp
from jax import lax
from jax.experimental import pallas as pl
from jax.experimental.pallas import tpu as pltpu
from jax.experimental.pallas import tpu_sc as plsc

B, L = 4096, 50
N = B * L
D_DATA, D_T, D_W = 64, 32, 32
D_OUT = D_DATA + D_T + D_W
NUM_CORES, NUM_SUBCORES = 2, 16
NW = NUM_CORES * NUM_SUBCORES
ROWS_PER_W = N // NW
CHUNK = 128
NCHUNK = ROWS_PER_W // CHUNK
NBUF = 4

_mesh = plsc.VectorSubcoreMesh(core_axis_name="c", subcore_axis_name="s")


@functools.partial(
    pl.kernel,
    mesh=_mesh,
    compiler_params=pltpu.CompilerParams(use_tc_tiling_on_sc=False),
    out_type=jax.ShapeDtypeStruct((N, D_OUT), jnp.float32),
    scratch_types=[
        pltpu.VMEM((NCHUNK, CHUNK), jnp.int32),
        pltpu.VMEM((NCHUNK, CHUNK), jnp.int32),
        pltpu.VMEM((NBUF, CHUNK, D_T), jnp.float32),
        pltpu.VMEM((NBUF, CHUNK, D_W), jnp.float32),
        pltpu.SemaphoreType.DMA((NBUF,)),
        pltpu.SemaphoreType.DMA((NBUF,)),
        pltpu.SemaphoreType.DMA((NBUF,)),
        pltpu.SemaphoreType.DMA((NBUF,)),
        pltpu.SemaphoreType.DMA,
    ],
)
def _embed_sc(data_hbm, time_hbm, wday_hbm, et_hbm, ew_hbm, out_hbm,
              tidx, widx, tbuf, wbuf, gsem_t, gsem_w, wsem_t, wsem_w, dsem):
    wid = lax.axis_index("s") * NUM_CORES + lax.axis_index("c")
    base0 = wid * ROWS_PER_W

    pltpu.sync_copy(time_hbm.at[wid], tidx)
    pltpu.sync_copy(wday_hbm.at[wid], widx)
    data_copy = pltpu.async_copy(
        data_hbm.at[pl.ds(base0, ROWS_PER_W), :],
        out_hbm.at[pl.ds(base0, ROWS_PER_W), pl.ds(0, D_DATA)],
        dsem)

    def fire_gathers(j, s):
        pltpu.async_copy(et_hbm.at[tidx.at[j]], tbuf.at[s], gsem_t.at[s])
        pltpu.async_copy(ew_hbm.at[widx.at[j]], wbuf.at[s], gsem_w.at[s])

    def drain_writes(s):
        pltpu.make_async_copy(
            tbuf.at[s], out_hbm.at[pl.ds(base0, CHUNK), pl.ds(D_DATA, D_T)],
            wsem_t.at[s]).wait()
        pltpu.make_async_copy(
            wbuf.at[s],
            out_hbm.at[pl.ds(base0, CHUNK), pl.ds(D_DATA + D_T, D_W)],
            wsem_w.at[s]).wait()

    def consume(j, s):
        base = base0 + j * CHUNK
        pltpu.make_async_copy(et_hbm.at[tidx.at[j]], tbuf.at[s],
                              gsem_t.at[s]).wait()
        pltpu.async_copy(tbuf.at[s],
                         out_hbm.at[pl.ds(base, CHUNK), pl.ds(D_DATA, D_T)],
                         wsem_t.at[s])
        pltpu.make_async_copy(ew_hbm.at[widx.at[j]], wbuf.at[s],
                              gsem_w.at[s]).wait()
        pltpu.async_copy(wbuf.at[s],
                         out_hbm.at[pl.ds(base, CHUNK), pl.ds(D_DATA + D_T, D_W)],
                         wsem_w.at[s])

    for s in range(NBUF):
        fire_gathers(s, s)

    def body(g, carry):
        for s in range(NBUF):
            j = g * NBUF + s
            consume(j, s)
            nxt = j + NBUF

            @pl.when(nxt < NCHUNK)
            def _():
                drain_writes(s)
                fire_gathers(nxt, s)
        return carry

    lax.fori_loop(0, NCHUNK // NBUF, body, 0)
    for j in range((NCHUNK // NBUF) * NBUF, NCHUNK):
        s = j % NBUF
        consume(j, s)
    for s in range(NBUF):
        drain_writes(s)
    data_copy.wait()


def kernel(data, time, weekday, emb_time, emb_weekday):
    d = data.reshape(N, D_DATA)
    t = time.reshape(NW, NCHUNK, CHUNK).astype(jnp.int32)
    w = weekday.reshape(NW, NCHUNK, CHUNK).astype(jnp.int32)
    out = _embed_sc(d, t, w, emb_time, emb_weekday)
    return out.reshape(B, L, D_OUT)


# speedup vs baseline: 1.2225x; 1.2225x over previous
import functools

import jax
import jax.numpy as jnp
from jax import lax
from jax.experimental import pallas as pl
from jax.experimental.pallas import tpu as pltpu
from jax.experimental.pallas import tpu_sc as plsc

B, L = 4096, 50
N = B * L
D_DATA, D_T, D_W = 64, 32, 32
D_OUT = D_DATA + D_T + D_W
NUM_CORES, NUM_SUBCORES = 2, 16
NW = NUM_CORES * NUM_SUBCORES
ROWS_PER_W = N // NW
CHUNK = 128
NCHUNK = ROWS_PER_W // CHUNK
NBUF = 4

_mesh = plsc.VectorSubcoreMesh(core_axis_name="c", subcore_axis_name="s")


@functools.partial(
    pl.kernel,
    mesh=_mesh,
    compiler_params=pltpu.CompilerParams(use_tc_tiling_on_sc=False),
    out_type=jax.ShapeDtypeStruct((N, D_OUT), jnp.float32),
    scratch_types=[
        pltpu.VMEM((NCHUNK, CHUNK), jnp.int32),
        pltpu.VMEM((NCHUNK, CHUNK), jnp.int32),
        pltpu.VMEM((NBUF, CHUNK, D_T), jnp.float32),
        pltpu.VMEM((NBUF, CHUNK, D_W), jnp.float32),
        pltpu.SemaphoreType.DMA((NBUF,)),
        pltpu.SemaphoreType.DMA((NBUF,)),
        pltpu.SemaphoreType.DMA((NBUF,)),
        pltpu.SemaphoreType.DMA((NBUF,)),
        pltpu.SemaphoreType.DMA,
    ],
)
def _embed_sc(data_hbm, time_hbm, wday_hbm, et_hbm, ew_hbm, out_hbm,
              tidx, widx, tbuf, wbuf, gsem_t, gsem_w, wsem_t, wsem_w, dsem):
    wid = lax.axis_index("s") * NUM_CORES + lax.axis_index("c")
    base0 = wid * ROWS_PER_W

    pltpu.sync_copy(time_hbm.at[wid], tidx)
    pltpu.sync_copy(wday_hbm.at[wid], widx)
    data_copy = None

    def fire_gathers(j, s):
        pltpu.async_copy(et_hbm.at[tidx.at[j]], tbuf.at[s], gsem_t.at[s])
        pltpu.async_copy(ew_hbm.at[widx.at[j]], wbuf.at[s], gsem_w.at[s])

    def drain_writes(s):
        pltpu.make_async_copy(
            tbuf.at[s], out_hbm.at[pl.ds(base0, CHUNK), pl.ds(D_DATA, D_T)],
            wsem_t.at[s]).wait()
        pltpu.make_async_copy(
            wbuf.at[s],
            out_hbm.at[pl.ds(base0, CHUNK), pl.ds(D_DATA + D_T, D_W)],
            wsem_w.at[s]).wait()

    def consume(j, s):
        base = base0 + j * CHUNK
        pltpu.make_async_copy(et_hbm.at[tidx.at[j]], tbuf.at[s],
                              gsem_t.at[s]).wait()
        pltpu.async_copy(tbuf.at[s],
                         out_hbm.at[pl.ds(base, CHUNK), pl.ds(D_DATA, D_T)],
                         wsem_t.at[s])
        pltpu.make_async_copy(ew_hbm.at[widx.at[j]], wbuf.at[s],
                              gsem_w.at[s]).wait()
        pltpu.async_copy(wbuf.at[s],
                         out_hbm.at[pl.ds(base, CHUNK), pl.ds(D_DATA + D_T, D_W)],
                         wsem_w.at[s])

    for s in range(NBUF):
        fire_gathers(s, s)

    def body(g, carry):
        for s in range(NBUF):
            j = g * NBUF + s
            consume(j, s)
            nxt = j + NBUF

            @pl.when(nxt < NCHUNK)
            def _():
                drain_writes(s)
                fire_gathers(nxt, s)
        return carry

    lax.fori_loop(0, NCHUNK // NBUF, body, 0)
    for j in range((NCHUNK // NBUF) * NBUF, NCHUNK):
        s = j % NBUF
        consume(j, s)
    for s in range(NBUF):
        drain_writes(s)
    if data_copy is not None:
        data_copy.wait()


def kernel(data, time, weekday, emb_time, emb_weekday):
    d = data.reshape(N, D_DATA)
    t = time.reshape(NW, NCHUNK, CHUNK).astype(jnp.int32)
    w = weekday.reshape(NW, NCHUNK, CHUNK).astype(jnp.int32)
    out = _embed_sc(d, t, w, emb_time, emb_weekday)
    return out.reshape(B, L, D_OUT)


# speedup vs baseline: 4.1491x; 3.3940x over previous
import functools

import jax
import jax.numpy as jnp
from jax import lax
from jax.experimental import pallas as pl
from jax.experimental.pallas import tpu as pltpu
from jax.experimental.pallas import tpu_sc as plsc

B, L = 4096, 50
N = B * L
D_DATA, D_T, D_W = 64, 32, 32
D_OUT = D_DATA + D_T + D_W
NUM_TIMES, NUM_WEEKDAYS = 288, 7
NUM_CORES, NUM_SUBCORES = 2, 16
NW = NUM_CORES * NUM_SUBCORES
ROWS_PER_W = N // NW
CHUNK = 256
NCHUNK = ROWS_PER_W // CHUNK
LANES = 16

_mesh = plsc.VectorSubcoreMesh(core_axis_name="c", subcore_axis_name="s")


@functools.partial(
    pl.kernel,
    mesh=_mesh,
    compiler_params=pltpu.CompilerParams(use_tc_tiling_on_sc=False,
                                         needs_layout_passes=False),
    out_type=jax.ShapeDtypeStruct((N, D_OUT), jnp.float32),
    scratch_types=[
        pltpu.VMEM((NUM_TIMES * D_T,), jnp.float32),
        pltpu.VMEM((NUM_WEEKDAYS * D_W,), jnp.float32),
        pltpu.VMEM((ROWS_PER_W,), jnp.int32),
        pltpu.VMEM((ROWS_PER_W,), jnp.int32),
        pltpu.VMEM((2, CHUNK, D_OUT), jnp.float32),
        pltpu.SemaphoreType.DMA((2,)),
        pltpu.SemaphoreType.DMA((2,)),
    ],
)
def _embed_sc(data_hbm, time_hbm, wday_hbm, et_hbm, ew_hbm, out_hbm,
              et_v, ew_v, tidx, widx, obuf, dsem, wsem):
    wid = lax.axis_index("s") * NUM_CORES + lax.axis_index("c")
    base0 = wid * ROWS_PER_W

    pltpu.sync_copy(et_hbm, et_v)
    pltpu.sync_copy(ew_hbm, ew_v)
    pltpu.sync_copy(time_hbm.at[wid], tidx)
    pltpu.sync_copy(wday_hbm.at[wid], widx)

    iota = lax.iota(jnp.int32, LANES)

    def do_chunk(co, s):
        coff = co * CHUNK
        gbase = base0 + coff

        @pl.when(co >= 2)
        def _():
            pltpu.make_async_copy(
                obuf.at[s], out_hbm.at[pl.ds(base0, CHUNK), :],
                wsem.at[s]).wait()

        data_cp = pltpu.async_copy(
            data_hbm.at[pl.ds(gbase, CHUNK), :],
            obuf.at[s].at[:, pl.ds(0, D_DATA)],
            dsem.at[s])

        def group(gr, carry):
            rbase = gr * LANES
            tv = tidx[pl.ds(coff + rbase, LANES)] * D_T
            wv = widx[pl.ds(coff + rbase, LANES)] * D_W
            for i in range(LANES):
                r = rbase + i
                tb = tv[i]
                wb = wv[i]
                obuf[s, r, pl.ds(D_DATA, LANES)] = plsc.load_gather(
                    et_v, [tb + iota])
                obuf[s, r, pl.ds(D_DATA + LANES, LANES)] = plsc.load_gather(
                    et_v, [tb + LANES + iota])
                obuf[s, r, pl.ds(D_DATA + D_T, LANES)] = plsc.load_gather(
                    ew_v, [wb + iota])
                obuf[s, r, pl.ds(D_DATA + D_T + LANES, LANES)] = plsc.load_gather(
                    ew_v, [wb + LANES + iota])
            return carry

        lax.fori_loop(0, CHUNK // LANES, group, 0)
        data_cp.wait()
        pltpu.async_copy(obuf.at[s], out_hbm.at[pl.ds(gbase, CHUNK), :],
                         wsem.at[s])

    def body(g, carry):
        do_chunk(g * 2, 0)
        do_chunk(g * 2 + 1, 1)
        return carry

    lax.fori_loop(0, NCHUNK // 2, body, 0)
    do_chunk(NCHUNK - 1, 0)

    for s in range(2):
        pltpu.make_async_copy(
            obuf.at[s], out_hbm.at[pl.ds(base0, CHUNK), :],
            wsem.at[s]).wait()


def kernel(data, time, weekday, emb_time, emb_weekday):
    d = data.reshape(N, D_DATA)
    t = time.reshape(NW, ROWS_PER_W).astype(jnp.int32)
    w = weekday.reshape(NW, ROWS_PER_W).astype(jnp.int32)
    out = _embed_sc(d, t, w,
                    emb_time.reshape(NUM_TIMES * D_T),
                    emb_weekday.reshape(NUM_WEEKDAYS * D_W))
    return out.reshape(B, L, D_OUT)
